# baseline (speedup 1.0000x reference)
import functools

import jax
import jax.numpy as jnp
from jax import lax
from jax.experimental import pallas as pl
from jax.experimental.pallas import tpu as pltpu
from jax.experimental.pallas import tpu_sc as plsc

B = 4096
S = 200
R = S + 1
NT = B * R
NW = 32
PW = NT // NW
CHUNK = 128
NCHUNK = PW // CHUNK
D = 64
CLS_ROW = 5162


def _sc_gather(table, src):
    mesh = plsc.VectorSubcoreMesh(core_axis_name="c", subcore_axis_name="s")

    @functools.partial(
        pl.kernel,
        out_type=jax.ShapeDtypeStruct((NT, D), jnp.float32),
        mesh=mesh,
        scratch_types=[
            pltpu.VMEM((CHUNK,), jnp.int32),
            pltpu.VMEM((CHUNK, D), jnp.float32),
            pltpu.SemaphoreType.DMA,
        ],
        compiler_params=pltpu.CompilerParams(use_tc_tiling_on_sc=False),
    )
    def sc_kernel(table_hbm, src_hbm, out_hbm, src_v, rows_v, g_sem):
        wid = lax.axis_index("s") * 2 + lax.axis_index("c")
        base = wid * PW

        @pl.loop(0, NCHUNK)
        def _(c):
            start = base + c * CHUNK
            pltpu.sync_copy(src_hbm.at[pl.ds(start, CHUNK)], src_v)
            pltpu.async_copy(table_hbm.at[src_v], rows_v, g_sem).wait()
            pltpu.sync_copy(rows_v, out_hbm.at[pl.ds(start, CHUNK)])

    return sc_kernel(table, src)


BB = 32
BBR = BB * R


def _tc_body(y_ref, xp_ref, l_ref, p_ref, b_ref, emb_ref, mask_ref):
    emb_ref[:, 0:D] = y_ref[...]
    emb_ref[:, D:2 * D] = (
        jnp.dot(xp_ref[...], p_ref[...], preferred_element_type=jnp.float32)
        + b_ref[...])
    pos = lax.broadcasted_iota(jnp.int32, (BB, R), 1)
    mask_ref[...] = (pos >= l_ref[...] + 1).astype(jnp.int8)


def _tc_pass(y, xp, l2, p, b2):
    return pl.pallas_call(
        _tc_body,
        grid=(B // BB,),
        in_specs=[
            pl.BlockSpec((BBR, D), lambda i: (i, 0)),
            pl.BlockSpec((BBR, 4), lambda i: (i, 0)),
            pl.BlockSpec((BB, 1), lambda i: (i, 0)),
            pl.BlockSpec((4, D), lambda i: (0, 0)),
            pl.BlockSpec((1, D), lambda i: (0, 0)),
        ],
        out_specs=[
            pl.BlockSpec((BBR, 2 * D), lambda i: (i, 0)),
            pl.BlockSpec((BB, R), lambda i: (i, 0)),
        ],
        out_shape=[
            jax.ShapeDtypeStruct((NT, 2 * D), jnp.float32),
            jax.ShapeDtypeStruct((B, R), jnp.int8),
        ],
    )(y, xp, l2, p, b2)


def kernel(x, l, dom_table, W, b, cls):
    dom_idx = x[:, :, 3].astype(jnp.int32)
    src = jnp.concatenate(
        [jnp.full((B, 1), CLS_ROW, jnp.int32), dom_idx], axis=1
    ).reshape(NT)
    table = jnp.concatenate([dom_table, cls[0, :, :D]], axis=0)
    y = _sc_gather(table, src)

    feat = jnp.concatenate(
        [x[:, :, :3], jnp.zeros((B, S, 1), jnp.float32)], axis=2)
    flag = jnp.broadcast_to(
        jnp.array([0.0, 0.0, 0.0, 1.0], jnp.float32), (B, 1, 4))
    xp = jnp.concatenate([flag, feat], axis=1).reshape(NT, 4)
    clshi = cls[0, :, D:]
    p = jnp.concatenate([W.T, clshi - b.reshape(1, D)], axis=0)

    emb, mask8 = _tc_pass(y, xp, l.reshape(B, 1), p, b.reshape(1, D))
    return emb.reshape(B, R, 2 * D), mask8.astype(jnp.bool_)


# speedup vs baseline: 1.1425x; 1.1425x over previous
import functools

import jax
import jax.numpy as jnp
from jax import lax
from jax.experimental import pallas as pl
from jax.experimental.pallas import tpu as pltpu
from jax.experimental.pallas import tpu_sc as plsc

B = 4096
S = 200
R = S + 1
NW = 32
ROWS_PW = B // NW
D = 64
CLS_ROW = 5162


def _sc_gather(table, src):
    mesh = plsc.VectorSubcoreMesh(core_axis_name="c", subcore_axis_name="s")

    @functools.partial(
        pl.kernel,
        out_type=jax.ShapeDtypeStruct((B, R, D), jnp.float32),
        mesh=mesh,
        scratch_types=[
            pltpu.VMEM((R,), jnp.int32),
            pltpu.VMEM((R, D), jnp.float32),
            pltpu.SemaphoreType.DMA,
        ],
        compiler_params=pltpu.CompilerParams(use_tc_tiling_on_sc=False),
    )
    def sc_kernel(table_hbm, src_hbm, out_hbm, idx_v, rows_v, g_sem):
        wid = lax.axis_index("s") * 2 + lax.axis_index("c")
        base = wid * ROWS_PW

        @pl.loop(0, ROWS_PW)
        def _(rr):
            row = base + rr
            pltpu.sync_copy(src_hbm.at[row], idx_v)
            pltpu.async_copy(table_hbm.at[idx_v], rows_v, g_sem).wait()
            pltpu.sync_copy(rows_v, out_hbm.at[row])

    return sc_kernel(table, src)


BB = 32


def _tc_body(y_ref, xp_ref, l_ref, p_ref, b_ref, emb_ref, mask_ref):
    xp = xp_ref[...]
    dense = (xp[:, :, 0:1] * p_ref[0:1, :][None]
             + xp[:, :, 1:2] * p_ref[1:2, :][None]
             + xp[:, :, 2:3] * p_ref[2:3, :][None]
             + xp[:, :, 3:4] * p_ref[3:4, :][None]
             + b_ref[0:1, :][None])
    emb_ref[:, :, 0:D] = y_ref[...]
    emb_ref[:, :, D:2 * D] = dense
    pos = lax.broadcasted_iota(jnp.int32, (BB, R), 1)
    mask_ref[...] = (pos >= l_ref[...] + 1).astype(jnp.int8)


def _tc_pass(y, xp, l2, p, b2):
    return pl.pallas_call(
        _tc_body,
        grid=(B // BB,),
        in_specs=[
            pl.BlockSpec((BB, R, D), lambda i: (i, 0, 0)),
            pl.BlockSpec((BB, R, 4), lambda i: (i, 0, 0)),
            pl.BlockSpec((BB, 1), lambda i: (i, 0)),
            pl.BlockSpec((4, D), lambda i: (0, 0)),
            pl.BlockSpec((1, D), lambda i: (0, 0)),
        ],
        out_specs=[
            pl.BlockSpec((BB, R, 2 * D), lambda i: (i, 0, 0)),
            pl.BlockSpec((BB, R), lambda i: (i, 0)),
        ],
        out_shape=[
            jax.ShapeDtypeStruct((B, R, 2 * D), jnp.float32),
            jax.ShapeDtypeStruct((B, R), jnp.int8),
        ],
    )(y, xp, l2, p, b2)


def kernel(x, l, dom_table, W, b, cls):
    dom_idx = x[:, :, 3].astype(jnp.int32)
    src = jnp.concatenate(
        [jnp.full((B, 1), CLS_ROW, jnp.int32), dom_idx], axis=1)
    table = jnp.concatenate([dom_table, cls[0, :, :D]], axis=0)
    y = _sc_gather(table, src)

    feat = jnp.concatenate(
        [x[:, :, :3], jnp.zeros((B, S, 1), jnp.float32)], axis=2)
    flag = jnp.broadcast_to(
        jnp.array([0.0, 0.0, 0.0, 1.0], jnp.float32), (B, 1, 4))
    xp = jnp.concatenate([flag, feat], axis=1)
    clshi = cls[0, :, D:]
    p = jnp.concatenate([W.T, clshi - b.reshape(1, D)], axis=0)

    emb, mask8 = _tc_pass(y, xp, l.reshape(B, 1), p, b.reshape(1, D))
    return emb, mask8.astype(jnp.bool_)


# speedup vs baseline: 1.2844x; 1.1242x over previous
import functools

import jax
import jax.numpy as jnp
from jax import lax
from jax.experimental import pallas as pl
from jax.experimental.pallas import tpu as pltpu
from jax.experimental.pallas import tpu_sc as plsc

B = 4096
S = 200
R = S + 1
NW = 32
ROWS_PW = B // NW
D = 64
CLS_ROW = 5162


def _sc_gather(table, src):
    mesh = plsc.VectorSubcoreMesh(core_axis_name="c", subcore_axis_name="s")

    @functools.partial(
        pl.kernel,
        out_type=jax.ShapeDtypeStruct((B, R, D), jnp.float32),
        mesh=mesh,
        scratch_types=[
            pltpu.VMEM((R,), jnp.int32),
            pltpu.VMEM((R, D), jnp.float32),
            pltpu.SemaphoreType.DMA,
        ],
        compiler_params=pltpu.CompilerParams(use_tc_tiling_on_sc=False),
    )
    def sc_kernel(table_hbm, src_hbm, out_hbm, idx_v, rows_v, g_sem):
        wid = lax.axis_index("s") * 2 + lax.axis_index("c")
        base = wid * ROWS_PW

        @pl.loop(0, ROWS_PW)
        def _(rr):
            row = base + rr
            pltpu.sync_copy(src_hbm.at[row], idx_v)
            pltpu.async_copy(table_hbm.at[idx_v], rows_v, g_sem).wait()
            pltpu.sync_copy(rows_v, out_hbm.at[row])

    return sc_kernel(table, src)


BB = 32


def _tc_body(y_ref, x_ref, l_ref, p_ref, b_ref, clshi_ref, emb_ref, mask_ref):
    dense = lax.dot_general(
        x_ref[...], p_ref[...],
        dimension_numbers=(((2,), (0,)), ((), ())),
        preferred_element_type=jnp.float32) + b_ref[0:1, :][None]
    emb_ref[:, :, 0:D] = y_ref[...]
    emb_ref[:, 0:1, D:2 * D] = jnp.broadcast_to(clshi_ref[...], (BB, 1, D))
    emb_ref[:, 1:, D:2 * D] = dense
    pos = lax.broadcasted_iota(jnp.int32, (BB, R), 1)
    mask_ref[...] = (pos >= l_ref[...] + 1).astype(jnp.int8)


def _tc_pass(y, x, l2, p, b2, clshi):
    return pl.pallas_call(
        _tc_body,
        grid=(B // BB,),
        in_specs=[
            pl.BlockSpec((BB, R, D), lambda i: (i, 0, 0)),
            pl.BlockSpec((BB, S, 4), lambda i: (i, 0, 0)),
            pl.BlockSpec((BB, 1), lambda i: (i, 0)),
            pl.BlockSpec((4, D), lambda i: (0, 0)),
            pl.BlockSpec((1, D), lambda i: (0, 0)),
            pl.BlockSpec((1, 1, D), lambda i: (0, 0, 0)),
        ],
        out_specs=[
            pl.BlockSpec((BB, R, 2 * D), lambda i: (i, 0, 0)),
            pl.BlockSpec((BB, R), lambda i: (i, 0)),
        ],
        out_shape=[
            jax.ShapeDtypeStruct((B, R, 2 * D), jnp.float32),
            jax.ShapeDtypeStruct((B, R), jnp.int8),
        ],
    )(y, x, l2, p, b2, clshi)


def kernel(x, l, dom_table, W, b, cls):
    dom_idx = x[:, :, 3].astype(jnp.int32)
    src = jnp.concatenate(
        [jnp.full((B, 1), CLS_ROW, jnp.int32), dom_idx], axis=1)
    table = jnp.concatenate([dom_table, cls[0, :, :D]], axis=0)
    y = _sc_gather(table, src)

    p = jnp.concatenate([W.T, jnp.zeros((1, D), jnp.float32)], axis=0)
    emb, mask8 = _tc_pass(y, x, l.reshape(B, 1), p, b.reshape(1, D),
                          cls[:, :, D:])
    return emb, mask8.astype(jnp.bool_)


# speedup vs baseline: 1.4451x; 1.1252x over previous
import functools

import jax
import jax.numpy as jnp
from jax import lax
from jax.experimental import pallas as pl
from jax.experimental.pallas import tpu as pltpu
from jax.experimental.pallas import tpu_sc as plsc

B = 4096
S = 200
R = S + 1
NW = 32
ROWS_PW = B // NW
D = 64
CLS_ROW = 5162


def _sc_gather(table, src):
    mesh = plsc.VectorSubcoreMesh(core_axis_name="c", subcore_axis_name="s")

    @functools.partial(
        pl.kernel,
        out_type=jax.ShapeDtypeStruct((B, R, D), jnp.float32),
        mesh=mesh,
        scratch_types=[
            pltpu.VMEM((R,), jnp.int32),
            pltpu.VMEM((R, D), jnp.float32),
            pltpu.SemaphoreType.DMA,
        ],
        compiler_params=pltpu.CompilerParams(use_tc_tiling_on_sc=False),
    )
    def sc_kernel(table_hbm, src_hbm, out_hbm, idx_v, rows_v, g_sem):
        wid = lax.axis_index("s") * 2 + lax.axis_index("c")
        base = wid * ROWS_PW

        @pl.loop(0, ROWS_PW)
        def _(rr):
            row = base + rr
            pltpu.sync_copy(src_hbm.at[row], idx_v)
            pltpu.async_copy(table_hbm.at[idx_v], rows_v, g_sem).wait()
            pltpu.sync_copy(rows_v, out_hbm.at[row])

    return sc_kernel(table, src)


BB = 32


def _tc_body(y_ref, x_ref, l_ref, p_ref, b_ref, clshi_ref, emb_ref, mask_ref):
    dense = lax.dot_general(
        x_ref[...], p_ref[...],
        dimension_numbers=(((1,), (0,)), ((), ())),
        preferred_element_type=jnp.float32) + b_ref[0:1, :][None]
    emb_ref[:, :, 0:D] = y_ref[...]
    emb_ref[:, 0:1, D:2 * D] = jnp.broadcast_to(clshi_ref[...], (BB, 1, D))
    emb_ref[:, 1:, D:2 * D] = dense
    pos = lax.broadcasted_iota(jnp.int32, (BB, R), 1)
    mask_ref[...] = (pos >= l_ref[...] + 1).astype(jnp.int8)


def _tc_pass(y, x, l2, p, b2, clshi):
    return pl.pallas_call(
        _tc_body,
        grid=(B // BB,),
        in_specs=[
            pl.BlockSpec((BB, R, D), lambda i: (i, 0, 0)),
            pl.BlockSpec((BB, 4, S), lambda i: (i, 0, 0)),
            pl.BlockSpec((BB, 1), lambda i: (i, 0)),
            pl.BlockSpec((4, D), lambda i: (0, 0)),
            pl.BlockSpec((1, D), lambda i: (0, 0)),
            pl.BlockSpec((1, 1, D), lambda i: (0, 0, 0)),
        ],
        out_specs=[
            pl.BlockSpec((BB, R, 2 * D), lambda i: (i, 0, 0)),
            pl.BlockSpec((BB, R), lambda i: (i, 0)),
        ],
        out_shape=[
            jax.ShapeDtypeStruct((B, R, 2 * D), jnp.float32),
            jax.ShapeDtypeStruct((B, R), jnp.int8),
        ],
    )(y, x, l2, p, b2, clshi)


def kernel(x, l, dom_table, W, b, cls):
    dom_idx = x[:, :, 3].astype(jnp.int32)
    src = jnp.concatenate(
        [jnp.full((B, 1), CLS_ROW, jnp.int32), dom_idx], axis=1)
    table = jnp.concatenate([dom_table, cls[0, :, :D]], axis=0)
    y = _sc_gather(table, src)

    p = jnp.concatenate([W.T, jnp.zeros((1, D), jnp.float32)], axis=0)
    xt = jnp.transpose(x, (0, 2, 1))
    emb, mask8 = _tc_pass(y, xt, l.reshape(B, 1), p, b.reshape(1, D),
                          cls[:, :, D:])
    return emb, mask8.astype(jnp.bool_)
